# baseline (speedup 1.0000x reference)
import functools

import numpy as np
import jax
import jax.numpy as jnp
from jax import lax
from jax.experimental import pallas as pl
from jax.experimental.pallas import tpu as pltpu
from jax.experimental.pallas import tpu_sc as plsc

_D = 64
_NC, _NS = 2, 16
_NW = _NC * _NS
_EPS = 1e-12
_TW = 2048


@functools.lru_cache(maxsize=None)
def _pe_const(seq_len: int):
    position = np.arange(0, seq_len, dtype=np.float32)[:, None]
    div_term = np.exp(np.arange(0, _D, 2, dtype=np.float32) * -(np.log(10000.0) / _D))
    pe = np.zeros((seq_len, _D), dtype=np.float32)
    pe[:, 0::2] = np.sin(position * div_term)
    pe[:, 1::2] = np.cos(position * div_term)
    return pe


def _tr_body(x1_ref, x2_ref, o_ref):
    o_ref[...] = jnp.concatenate([x1_ref[...].T, x2_ref[...].T], axis=1)


@functools.lru_cache(maxsize=None)
def _tr_call(vocab: int, half: int):
    grid = half // _TW
    last = (vocab - 1) // _TW
    return pl.pallas_call(
        _tr_body,
        grid=(grid,),
        in_specs=[
            pl.BlockSpec((_D, _TW), lambda i: (0, i)),
            pl.BlockSpec((_D, _TW), lambda i: (0, jnp.minimum(i + grid, last))),
        ],
        out_specs=pl.BlockSpec((_TW, 2 * _D), lambda i: (i, 0)),
        out_shape=jax.ShapeDtypeStruct((half, 2 * _D), jnp.float32),
    )


@functools.lru_cache(maxsize=None)
def _gather_call(n_rows: int, table_rows: int, chunk: int):
    assert n_rows % (_NW * chunk) == 0 and chunk % 8 == 0
    n_chunks = n_rows // (_NW * chunk)
    rows_per_w = n_chunks * chunk
    mesh = plsc.VectorSubcoreMesh(
        core_axis_name="c", subcore_axis_name="s",
        num_cores=_NC, num_subcores=_NS)

    @functools.partial(
        pl.kernel,
        out_type=jax.ShapeDtypeStruct((n_rows, _D), jnp.float32),
        mesh=mesh,
        scratch_types=[
            pltpu.VMEM((chunk,), jnp.int32),
            pltpu.VMEM((chunk, _D), jnp.float32),
            pltpu.SemaphoreType.DMA,
        ],
        compiler_params=pltpu.CompilerParams(use_tc_tiling_on_sc=False),
    )
    def k(idx_hbm, table_hbm, out_hbm, idx_v, rows_v, sem):
        wid = lax.axis_index("s") * _NC + lax.axis_index("c")
        base0 = wid * rows_per_w

        def body(i, carry):
            base = base0 + i * chunk
            pltpu.sync_copy(idx_hbm.at[pl.ds(base, chunk)], idx_v)
            pltpu.async_copy(table_hbm.at[idx_v], rows_v, sem).wait()
            pltpu.sync_copy(rows_v, out_hbm.at[pl.ds(base, chunk)])
            return carry

        lax.fori_loop(0, n_chunks, body, 0, unroll=False)

    return k


def _ln2_body(x_ref, pe_ref, g_ref, b_ref, o_ref):
    x = x_ref[...] + pe_ref[...]
    bb = x.shape[0]
    xl, xh = x[:, :_D], x[:, _D:]
    ml = jnp.mean(xl, axis=-1, keepdims=True)
    mh = jnp.mean(xh, axis=-1, keepdims=True)
    mu = jnp.concatenate(
        [jnp.broadcast_to(ml, (bb, _D)), jnp.broadcast_to(mh, (bb, _D))], -1)
    xc = x - mu
    sq = xc * xc
    vl = jnp.mean(sq[:, :_D], axis=-1, keepdims=True)
    vh = jnp.mean(sq[:, _D:], axis=-1, keepdims=True)
    var = jnp.concatenate(
        [jnp.broadcast_to(vl, (bb, _D)), jnp.broadcast_to(vh, (bb, _D))], -1)
    o_ref[...] = xc * lax.rsqrt(var + _EPS) * g_ref[...] + b_ref[...]


@functools.lru_cache(maxsize=None)
def _ln2_call(n_pairs: int, block: int):
    grid = (n_pairs // block,)
    return pl.pallas_call(
        _ln2_body,
        grid=grid,
        in_specs=[
            pl.BlockSpec((block, 2 * _D), lambda i: (i, 0)),
            pl.BlockSpec((block, 2 * _D), lambda i: (0, 0)),
            pl.BlockSpec((1, 2 * _D), lambda i: (0, 0)),
            pl.BlockSpec((1, 2 * _D), lambda i: (0, 0)),
        ],
        out_specs=pl.BlockSpec((block, 2 * _D), lambda i: (i, 0)),
        out_shape=jax.ShapeDtypeStruct((n_pairs, 2 * _D), jnp.float32),
    )


def kernel(input_ids, table, gamma, beta):
    B, S = input_ids.shape
    n = B * S
    V = table.shape[0]
    half = ((V // 2) // _TW + 1) * _TW

    t2 = _tr_call(V, half)(table.T, table.T)
    t_lin = t2.reshape(2 * half, _D)

    ids = input_ids.reshape(-1).astype(jnp.int32)
    idx = jnp.where(ids < half, 2 * ids, 2 * (ids - half) + 1)

    emb = _gather_call(n, 2 * half, 800)(idx, t_lin)
    emb2 = emb.reshape(n // 2, 2 * _D)

    block = 1600
    pe2 = jnp.asarray(_pe_const(S)).reshape(S // 2, 2 * _D)
    pe_blk = jnp.tile(pe2, (block // (S // 2), 1))
    g2 = jnp.concatenate([gamma, gamma]).reshape(1, 2 * _D)
    b2 = jnp.concatenate([beta, beta]).reshape(1, 2 * _D)
    out2 = _ln2_call(n // 2, block)(emb2, pe_blk, g2, b2)
    return out2.reshape(B, S, _D)


# speedup vs baseline: 1.0855x; 1.0855x over previous
import functools

import numpy as np
import jax
import jax.numpy as jnp
from jax import lax
from jax.experimental import pallas as pl
from jax.experimental.pallas import tpu as pltpu
from jax.experimental.pallas import tpu_sc as plsc

_D = 64
_NC, _NS = 2, 16
_NW = _NC * _NS
_EPS = 1e-12
_TW = 4096


@functools.lru_cache(maxsize=None)
def _pe_const(seq_len: int):
    position = np.arange(0, seq_len, dtype=np.float32)[:, None]
    div_term = np.exp(np.arange(0, _D, 2, dtype=np.float32) * -(np.log(10000.0) / _D))
    pe = np.zeros((seq_len, _D), dtype=np.float32)
    pe[:, 0::2] = np.sin(position * div_term)
    pe[:, 1::2] = np.cos(position * div_term)
    return pe


def _tr_body(x1_ref, x2_ref, o_ref):
    o_ref[...] = jnp.concatenate([x1_ref[...].T, x2_ref[...].T], axis=1)


@functools.lru_cache(maxsize=None)
def _tr_call(vocab: int, half: int):
    grid = half // _TW
    last = (vocab - 1) // _TW
    return pl.pallas_call(
        _tr_body,
        grid=(grid,),
        in_specs=[
            pl.BlockSpec((_D, _TW), lambda i: (0, i)),
            pl.BlockSpec((_D, _TW), lambda i: (0, jnp.minimum(i + grid, last))),
        ],
        out_specs=pl.BlockSpec((_TW, 2 * _D), lambda i: (i, 0)),
        out_shape=jax.ShapeDtypeStruct((half, 2 * _D), jnp.float32),
    )


@functools.lru_cache(maxsize=None)
def _gather_call(n_rows: int, table_rows: int, chunk: int):
    assert n_rows % (_NW * chunk) == 0 and chunk % 8 == 0
    n_chunks = n_rows // (_NW * chunk)
    rows_per_w = n_chunks * chunk
    mesh = plsc.VectorSubcoreMesh(
        core_axis_name="c", subcore_axis_name="s",
        num_cores=_NC, num_subcores=_NS)

    @functools.partial(
        pl.kernel,
        out_type=jax.ShapeDtypeStruct((n_rows, _D), jnp.float32),
        mesh=mesh,
        scratch_types=[
            pltpu.VMEM((chunk,), jnp.int32),
            pltpu.VMEM((chunk, _D), jnp.float32),
            pltpu.SemaphoreType.DMA,
        ],
        compiler_params=pltpu.CompilerParams(use_tc_tiling_on_sc=False),
    )
    def k(idx_hbm, table_hbm, out_hbm, idx_v, rows_v, sem):
        wid = lax.axis_index("s") * _NC + lax.axis_index("c")
        base0 = wid * rows_per_w

        def body(i, carry):
            base = base0 + i * chunk
            pltpu.sync_copy(idx_hbm.at[pl.ds(base, chunk)], idx_v)
            pltpu.async_copy(table_hbm.at[idx_v], rows_v, sem).wait()
            pltpu.sync_copy(rows_v, out_hbm.at[pl.ds(base, chunk)])
            return carry

        lax.fori_loop(0, n_chunks, body, 0, unroll=False)

    return k


def _ln2_body(x_ref, pe_ref, g_ref, b_ref, o_ref):
    x = x_ref[...] + pe_ref[...]
    lane = lax.broadcasted_iota(jnp.int32, (2 * _D, 2), 0)
    col = lax.broadcasted_iota(jnp.int32, (2 * _D, 2), 1)
    sel = (lane // _D) == col
    m = jnp.where(sel, 1.0 / _D, 0.0)
    p = jnp.where(sel, 1.0, 0.0).T
    mu = jnp.dot(jnp.dot(x, m, preferred_element_type=jnp.float32), p,
                 preferred_element_type=jnp.float32)
    xc = x - mu
    var = jnp.dot(jnp.dot(xc * xc, m, preferred_element_type=jnp.float32), p,
                  preferred_element_type=jnp.float32)
    o_ref[...] = xc * lax.rsqrt(var + _EPS) * g_ref[...] + b_ref[...]


@functools.lru_cache(maxsize=None)
def _ln2_call(n_pairs: int, block: int):
    grid = (n_pairs // block,)
    return pl.pallas_call(
        _ln2_body,
        grid=grid,
        in_specs=[
            pl.BlockSpec((block, 2 * _D), lambda i: (i, 0)),
            pl.BlockSpec((block, 2 * _D), lambda i: (0, 0)),
            pl.BlockSpec((1, 2 * _D), lambda i: (0, 0)),
            pl.BlockSpec((1, 2 * _D), lambda i: (0, 0)),
        ],
        out_specs=pl.BlockSpec((block, 2 * _D), lambda i: (i, 0)),
        out_shape=jax.ShapeDtypeStruct((n_pairs, 2 * _D), jnp.float32),
    )


def kernel(input_ids, table, gamma, beta):
    B, S = input_ids.shape
    n = B * S
    V = table.shape[0]
    half = ((V // 2) // _TW + 1) * _TW

    t2 = _tr_call(V, half)(table.T, table.T)
    t_lin = t2.reshape(2 * half, _D)

    ids = input_ids.reshape(-1).astype(jnp.int32)
    idx = jnp.where(ids < half, 2 * ids, 2 * (ids - half) + 1)

    emb = _gather_call(n, 2 * half, 800)(idx, t_lin)
    emb2 = emb.reshape(n // 2, 2 * _D)

    block = 800
    pe2 = jnp.asarray(_pe_const(S)).reshape(S // 2, 2 * _D)
    pe_blk = jnp.tile(pe2, (block // (S // 2), 1))
    g2 = jnp.concatenate([gamma, gamma]).reshape(1, 2 * _D)
    b2 = jnp.concatenate([beta, beta]).reshape(1, 2 * _D)
    out2 = _ln2_call(n // 2, block)(emb2, pe_blk, g2, b2)
    return out2.reshape(B, S, _D)


# speedup vs baseline: 1.4063x; 1.2956x over previous
import functools

import numpy as np
import jax
import jax.numpy as jnp
from jax import lax
from jax.experimental import pallas as pl
from jax.experimental.pallas import tpu as pltpu
from jax.experimental.pallas import tpu_sc as plsc

_D = 64
_NC, _NS = 2, 16
_NW = _NC * _NS
_EPS = 1e-12
_TW = 4096


@functools.lru_cache(maxsize=None)
def _pe_const(seq_len: int):
    position = np.arange(0, seq_len, dtype=np.float32)[:, None]
    div_term = np.exp(np.arange(0, _D, 2, dtype=np.float32) * -(np.log(10000.0) / _D))
    pe = np.zeros((seq_len, _D), dtype=np.float32)
    pe[:, 0::2] = np.sin(position * div_term)
    pe[:, 1::2] = np.cos(position * div_term)
    return pe


def _tr_body(x1_ref, x2_ref, o_ref):
    o_ref[...] = jnp.concatenate([x1_ref[...].T, x2_ref[...].T], axis=1)


@functools.lru_cache(maxsize=None)
def _tr_call(vocab: int, half: int):
    grid = half // _TW
    last = (vocab - 1) // _TW
    return pl.pallas_call(
        _tr_body,
        grid=(grid,),
        in_specs=[
            pl.BlockSpec((_D, _TW), lambda i: (0, i)),
            pl.BlockSpec((_D, _TW), lambda i: (0, jnp.minimum(i + grid, last))),
        ],
        out_specs=pl.BlockSpec((_TW, 2 * _D), lambda i: (i, 0)),
        out_shape=jax.ShapeDtypeStruct((half, 2 * _D), jnp.float32),
    )


@functools.lru_cache(maxsize=None)
def _gather_call(n_rows: int, table_rows: int, chunk: int):
    assert n_rows % (_NW * chunk) == 0 and chunk % 8 == 0
    n_chunks = n_rows // (_NW * chunk)
    rows_per_w = n_chunks * chunk
    mesh = plsc.VectorSubcoreMesh(
        core_axis_name="c", subcore_axis_name="s",
        num_cores=_NC, num_subcores=_NS)

    @functools.partial(
        pl.kernel,
        out_type=jax.ShapeDtypeStruct((n_rows, _D), jnp.float32),
        mesh=mesh,
        scratch_types=[
            pltpu.VMEM((chunk,), jnp.int32),
            pltpu.VMEM((chunk, _D), jnp.float32),
            pltpu.SemaphoreType.DMA,
        ],
        compiler_params=pltpu.CompilerParams(use_tc_tiling_on_sc=False),
    )
    def k(idx_hbm, table_hbm, out_hbm, idx_v, rows_v, sem):
        wid = lax.axis_index("s") * _NC + lax.axis_index("c")
        base0 = wid * rows_per_w

        def body(i, carry):
            base = base0 + i * chunk
            pltpu.sync_copy(idx_hbm.at[pl.ds(base, chunk)], idx_v)
            pltpu.async_copy(table_hbm.at[idx_v], rows_v, sem).wait()
            pltpu.sync_copy(rows_v, out_hbm.at[pl.ds(base, chunk)])
            return carry

        lax.fori_loop(0, n_chunks, body, 0, unroll=False)

    return k


def _lnt_body(x_ref, pe_ref, g_ref, b_ref, o_ref):
    x = x_ref[...] + pe_ref[...]
    lane = lax.broadcasted_iota(jnp.int32, (2 * _D, 2), 0)
    col = lax.broadcasted_iota(jnp.int32, (2 * _D, 2), 1)
    sel = (lane // _D) == col
    m = jnp.where(sel, 1.0 / _D, 0.0)
    p = jnp.where(sel, 1.0, 0.0).T
    mu = jnp.dot(jnp.dot(x, m, preferred_element_type=jnp.float32), p,
                 preferred_element_type=jnp.float32)
    xc = x - mu
    var = jnp.dot(jnp.dot(xc * xc, m, preferred_element_type=jnp.float32), p,
                  preferred_element_type=jnp.float32)
    y = xc * lax.rsqrt(var + _EPS) * g_ref[...] + b_ref[...]
    s2_len = y.shape[1]
    for s2 in range(s2_len):
        t = y[:, s2, :].T.reshape(2, 8, 8, 128)
        o_ref[2 * s2, :, 0, :, :] = t[0]
        o_ref[2 * s2 + 1, :, 0, :, :] = t[1]


@functools.lru_cache(maxsize=None)
def _lnt_call(batch: int, seq: int):
    s2 = seq // 2
    grid = (batch // 128,)
    return pl.pallas_call(
        _lnt_body,
        grid=grid,
        in_specs=[
            pl.BlockSpec((128, s2, 2 * _D), lambda i: (i, 0, 0)),
            pl.BlockSpec((1, s2, 2 * _D), lambda i: (0, 0, 0)),
            pl.BlockSpec((1, 1, 2 * _D), lambda i: (0, 0, 0)),
            pl.BlockSpec((1, 1, 2 * _D), lambda i: (0, 0, 0)),
        ],
        out_specs=pl.BlockSpec((seq, 8, 1, 8, 128), lambda i: (0, 0, i, 0, 0)),
        out_shape=jax.ShapeDtypeStruct(
            (seq, 8, batch // 128, 8, 128), jnp.float32),
        compiler_params=pltpu.CompilerParams(vmem_limit_bytes=100 * 1024 * 1024),
    )


def kernel(input_ids, table, gamma, beta):
    B, S = input_ids.shape
    n = B * S
    V = table.shape[0]
    half = ((V // 2) // _TW + 1) * _TW

    t2 = _tr_call(V, half)(table.T, table.T)
    t_lin = t2.reshape(2 * half, _D)

    ids = input_ids.reshape(-1).astype(jnp.int32)
    idx = jnp.where(ids < half, 2 * ids, 2 * (ids - half) + 1)

    emb = _gather_call(n, 2 * half, 800)(idx, t_lin)
    emb3 = emb.reshape(B, S // 2, 2 * _D)

    pe3 = jnp.asarray(_pe_const(S)).reshape(1, S // 2, 2 * _D)
    g2 = jnp.concatenate([gamma, gamma]).reshape(1, 1, 2 * _D)
    b2 = jnp.concatenate([beta, beta]).reshape(1, 1, 2 * _D)
    out_t = _lnt_call(B, S)(emb3, pe3, g2, b2)
    return out_t.transpose(2, 4, 0, 1, 3).reshape(B, S, _D)


# speedup vs baseline: 1.5506x; 1.1026x over previous
import functools

import numpy as np
import jax
import jax.numpy as jnp
from jax import lax
from jax.experimental import pallas as pl
from jax.experimental.pallas import tpu as pltpu
from jax.experimental.pallas import tpu_sc as plsc

_D = 64
_NC, _NS = 2, 16
_NW = _NC * _NS
_EPS = 1e-12
_TW = 4096


@functools.lru_cache(maxsize=None)
def _pe_const(seq_len: int):
    position = np.arange(0, seq_len, dtype=np.float32)[:, None]
    div_term = np.exp(np.arange(0, _D, 2, dtype=np.float32) * -(np.log(10000.0) / _D))
    pe = np.zeros((seq_len, _D), dtype=np.float32)
    pe[:, 0::2] = np.sin(position * div_term)
    pe[:, 1::2] = np.cos(position * div_term)
    return pe


def _tr_body(x1_ref, x2_ref, o_ref):
    o_ref[...] = jnp.concatenate([x1_ref[...].T, x2_ref[...].T], axis=1)


@functools.lru_cache(maxsize=None)
def _tr_call(vocab: int, half: int):
    grid = half // _TW
    last = (vocab - 1) // _TW
    return pl.pallas_call(
        _tr_body,
        grid=(grid,),
        in_specs=[
            pl.BlockSpec((_D, _TW), lambda i: (0, i)),
            pl.BlockSpec((_D, _TW), lambda i: (0, jnp.minimum(i + grid, last))),
        ],
        out_specs=pl.BlockSpec((_TW, 2 * _D), lambda i: (i, 0)),
        out_shape=jax.ShapeDtypeStruct((half, 2 * _D), jnp.float32),
    )


@functools.lru_cache(maxsize=None)
def _gather_call(n_rows: int, table_rows: int, chunk: int):
    assert n_rows % (_NW * chunk) == 0 and chunk % 8 == 0
    n_chunks = n_rows // (_NW * chunk)
    rows_per_w = n_chunks * chunk
    mesh = plsc.VectorSubcoreMesh(
        core_axis_name="c", subcore_axis_name="s",
        num_cores=_NC, num_subcores=_NS)

    @functools.partial(
        pl.kernel,
        out_type=jax.ShapeDtypeStruct((n_rows, _D), jnp.float32),
        mesh=mesh,
        scratch_types=[
            pltpu.VMEM((chunk,), jnp.int32),
            pltpu.VMEM((chunk, _D), jnp.float32),
            pltpu.SemaphoreType.DMA,
        ],
        compiler_params=pltpu.CompilerParams(use_tc_tiling_on_sc=False),
    )
    def k(idx_hbm, table_hbm, out_hbm, idx_v, rows_v, sem):
        wid = lax.axis_index("s") * _NC + lax.axis_index("c")
        base0 = wid * rows_per_w

        def body(i, carry):
            base = base0 + i * chunk
            pltpu.sync_copy(idx_hbm.at[pl.ds(base, chunk)], idx_v)
            pltpu.async_copy(table_hbm.at[idx_v], rows_v, sem).wait()
            pltpu.sync_copy(rows_v, out_hbm.at[pl.ds(base, chunk)])
            return carry

        lax.fori_loop(0, n_chunks, body, 0, unroll=False)

    return k


def _lnt_body(x_ref, pe_ref, g_ref, b_ref, o_ref):
    s2_len = pe_ref.shape[1]
    x = x_ref[...].reshape(128, s2_len, 2 * _D) + pe_ref[...]
    lane = lax.broadcasted_iota(jnp.int32, (2 * _D, 2), 0)
    col = lax.broadcasted_iota(jnp.int32, (2 * _D, 2), 1)
    sel = (lane // _D) == col
    m = jnp.where(sel, 1.0 / _D, 0.0)
    p = jnp.where(sel, 1.0, 0.0).T
    mu = jnp.dot(jnp.dot(x, m, preferred_element_type=jnp.float32), p,
                 preferred_element_type=jnp.float32)
    xc = x - mu
    var = jnp.dot(jnp.dot(xc * xc, m, preferred_element_type=jnp.float32), p,
                  preferred_element_type=jnp.float32)
    y = xc * lax.rsqrt(var + _EPS) * g_ref[...] + b_ref[...]
    for s2 in range(s2_len):
        t = y[:, s2, :].T.reshape(2, 8, 8, 128)
        o_ref[2 * s2, :, 0, :, :] = t[0]
        o_ref[2 * s2 + 1, :, 0, :, :] = t[1]


@functools.lru_cache(maxsize=None)
def _lnt_call(batch: int, seq: int):
    s2 = seq // 2
    grid = (batch // 128,)
    return pl.pallas_call(
        _lnt_body,
        grid=grid,
        in_specs=[
            pl.BlockSpec((128 * s2, 2 * _D), lambda i: (i, 0)),
            pl.BlockSpec((1, s2, 2 * _D), lambda i: (0, 0, 0)),
            pl.BlockSpec((1, 1, 2 * _D), lambda i: (0, 0, 0)),
            pl.BlockSpec((1, 1, 2 * _D), lambda i: (0, 0, 0)),
        ],
        out_specs=pl.BlockSpec((seq, 8, 1, 8, 128), lambda i: (0, 0, i, 0, 0)),
        out_shape=jax.ShapeDtypeStruct(
            (seq, 8, batch // 128, 8, 128), jnp.float32),
        compiler_params=pltpu.CompilerParams(vmem_limit_bytes=100 * 1024 * 1024),
    )


def kernel(input_ids, table, gamma, beta):
    B, S = input_ids.shape
    n = B * S
    V = table.shape[0]
    half = ((V // 2) // _TW + 1) * _TW

    t2 = _tr_call(V, half)(table.T, table.T)
    t_lin = t2.reshape(2 * half, _D)

    ids = input_ids.reshape(-1).astype(jnp.int32)
    idx = jnp.where(ids < half, 2 * ids, 2 * (ids - half) + 1)

    emb = _gather_call(n, 2 * half, 800)(idx, t_lin)
    emb2 = emb.reshape(n // 2, 2 * _D)

    pe3 = jnp.asarray(_pe_const(S)).reshape(1, S // 2, 2 * _D)
    g2 = jnp.concatenate([gamma, gamma]).reshape(1, 1, 2 * _D)
    b2 = jnp.concatenate([beta, beta]).reshape(1, 1, 2 * _D)
    out_t = _lnt_call(B, S)(emb2, pe3, g2, b2)
    return out_t.transpose(2, 4, 0, 1, 3).reshape(B, S, _D)


# speedup vs baseline: 1.6886x; 1.0890x over previous
import functools

import numpy as np
import jax
import jax.numpy as jnp
from jax import lax
from jax.experimental import pallas as pl
from jax.experimental.pallas import tpu as pltpu
from jax.experimental.pallas import tpu_sc as plsc

_D = 64
_NC, _NS = 2, 16
_NW = _NC * _NS
_EPS = 1e-12
_TW = 8192


@functools.lru_cache(maxsize=None)
def _pe_const(seq_len: int):
    position = np.arange(0, seq_len, dtype=np.float32)[:, None]
    div_term = np.exp(np.arange(0, _D, 2, dtype=np.float32) * -(np.log(10000.0) / _D))
    pe = np.zeros((seq_len, _D), dtype=np.float32)
    pe[:, 0::2] = np.sin(position * div_term)
    pe[:, 1::2] = np.cos(position * div_term)
    return pe


def _tr_body(x1_ref, x2_ref, o_ref):
    o_ref[...] = jnp.concatenate([x1_ref[...].T, x2_ref[...].T], axis=1)


@functools.lru_cache(maxsize=None)
def _tr_call(vocab: int, half: int):
    grid = half // _TW
    last = (vocab - 1) // _TW
    return pl.pallas_call(
        _tr_body,
        grid=(grid,),
        in_specs=[
            pl.BlockSpec((_D, _TW), lambda i: (0, i)),
            pl.BlockSpec((_D, _TW), lambda i: (0, jnp.minimum(i + grid, last))),
        ],
        out_specs=pl.BlockSpec((_TW, 2 * _D), lambda i: (i, 0)),
        out_shape=jax.ShapeDtypeStruct((half, 2 * _D), jnp.float32),
    )


@functools.lru_cache(maxsize=None)
def _gather_call(n_rows: int, table_rows: int, chunk: int):
    assert n_rows % (_NW * chunk) == 0 and chunk % 8 == 0
    n_chunks = n_rows // (_NW * chunk)
    rows_per_w = n_chunks * chunk
    mesh = plsc.VectorSubcoreMesh(
        core_axis_name="c", subcore_axis_name="s",
        num_cores=_NC, num_subcores=_NS)

    @functools.partial(
        pl.kernel,
        out_type=jax.ShapeDtypeStruct((n_rows, _D), jnp.float32),
        mesh=mesh,
        scratch_types=[
            pltpu.VMEM((chunk,), jnp.int32),
            pltpu.VMEM((chunk, _D), jnp.float32),
            pltpu.SemaphoreType.DMA,
        ],
        compiler_params=pltpu.CompilerParams(use_tc_tiling_on_sc=False),
    )
    def k(idx_hbm, table_hbm, out_hbm, idx_v, rows_v, sem):
        wid = lax.axis_index("s") * _NC + lax.axis_index("c")
        base0 = wid * rows_per_w

        def body(i, carry):
            base = base0 + i * chunk
            pltpu.sync_copy(idx_hbm.at[pl.ds(base, chunk)], idx_v)
            pltpu.async_copy(table_hbm.at[idx_v], rows_v, sem).wait()
            pltpu.sync_copy(rows_v, out_hbm.at[pl.ds(base, chunk)])
            return carry

        lax.fori_loop(0, n_chunks, body, 0, unroll=False)

    return k


def _lnt_body(x_ref, pe_ref, g_ref, b_ref, o_ref):
    s2_len = pe_ref.shape[1]
    x = x_ref[...].reshape(128, s2_len, 2 * _D) + pe_ref[...]
    lane = lax.broadcasted_iota(jnp.int32, (2 * _D, 2), 0)
    col = lax.broadcasted_iota(jnp.int32, (2 * _D, 2), 1)
    sel = (lane // _D) == col
    m = jnp.where(sel, 1.0 / _D, 0.0)
    p = jnp.where(sel, 1.0, 0.0).T
    mu = jnp.dot(jnp.dot(x, m, preferred_element_type=jnp.float32), p,
                 preferred_element_type=jnp.float32)
    xc = x - mu
    var = jnp.dot(jnp.dot(xc * xc, m, preferred_element_type=jnp.float32), p,
                  preferred_element_type=jnp.float32)
    y = xc * lax.rsqrt(var + _EPS) * g_ref[...] + b_ref[...]
    for s2 in range(s2_len):
        t = y[:, s2, :].T.reshape(2, 8, 8, 128)
        o_ref[2 * s2, :, 0, :, :] = t[0]
        o_ref[2 * s2 + 1, :, 0, :, :] = t[1]


@functools.lru_cache(maxsize=None)
def _lnt_call(batch: int, seq: int):
    s2 = seq // 2
    grid = (batch // 128,)
    return pl.pallas_call(
        _lnt_body,
        grid=grid,
        in_specs=[
            pl.BlockSpec((128 * s2, 2 * _D), lambda i: (i, 0)),
            pl.BlockSpec((1, s2, 2 * _D), lambda i: (0, 0, 0)),
            pl.BlockSpec((1, 1, 2 * _D), lambda i: (0, 0, 0)),
            pl.BlockSpec((1, 1, 2 * _D), lambda i: (0, 0, 0)),
        ],
        out_specs=pl.BlockSpec((seq, 8, 1, 8, 128), lambda i: (0, 0, i, 0, 0)),
        out_shape=jax.ShapeDtypeStruct(
            (seq, 8, batch // 128, 8, 128), jnp.float32),
        compiler_params=pltpu.CompilerParams(vmem_limit_bytes=100 * 1024 * 1024),
    )


def kernel(input_ids, table, gamma, beta):
    B, S = input_ids.shape
    n = B * S
    V = table.shape[0]
    half = ((V // 2) // _TW + 1) * _TW

    t2 = _tr_call(V, half)(table.T, table.T)
    t_lin = t2.reshape(2 * half, _D)

    ids = input_ids.reshape(-1).astype(jnp.int32)
    idx = jnp.where(ids < half, 2 * ids, 2 * (ids - half) + 1)

    emb = _gather_call(n, 2 * half, 800)(idx, t_lin)
    emb2 = emb.reshape(n // 2, 2 * _D)

    pe3 = jnp.asarray(_pe_const(S)).reshape(1, S // 2, 2 * _D)
    g2 = jnp.concatenate([gamma, gamma]).reshape(1, 1, 2 * _D)
    b2 = jnp.concatenate([beta, beta]).reshape(1, 1, 2 * _D)
    out_t = _lnt_call(B, S)(emb2, pe3, g2, b2)
    return out_t.transpose(2, 4, 0, 1, 3).reshape(B, S, _D)


# speedup vs baseline: 1.7590x; 1.0417x over previous
import functools

import numpy as np
import jax
import jax.numpy as jnp
from jax import lax
from jax.experimental import pallas as pl
from jax.experimental.pallas import tpu as pltpu
from jax.experimental.pallas import tpu_sc as plsc

_D = 64
_NC, _NS = 2, 16
_NW = _NC * _NS
_EPS = 1e-12
_TW = 16384


@functools.lru_cache(maxsize=None)
def _pe_const(seq_len: int):
    position = np.arange(0, seq_len, dtype=np.float32)[:, None]
    div_term = np.exp(np.arange(0, _D, 2, dtype=np.float32) * -(np.log(10000.0) / _D))
    pe = np.zeros((seq_len, _D), dtype=np.float32)
    pe[:, 0::2] = np.sin(position * div_term)
    pe[:, 1::2] = np.cos(position * div_term)
    return pe


def _tr_body(x1_ref, x2_ref, o_ref):
    o_ref[...] = jnp.concatenate([x1_ref[...].T, x2_ref[...].T], axis=1)


@functools.lru_cache(maxsize=None)
def _tr_call(vocab: int, half: int):
    grid = half // _TW
    last = (vocab - 1) // _TW
    return pl.pallas_call(
        _tr_body,
        grid=(grid,),
        in_specs=[
            pl.BlockSpec((_D, _TW), lambda i: (0, i)),
            pl.BlockSpec((_D, _TW), lambda i: (0, jnp.minimum(i + grid, last))),
        ],
        out_specs=pl.BlockSpec((_TW, 2 * _D), lambda i: (i, 0)),
        out_shape=jax.ShapeDtypeStruct((half, 2 * _D), jnp.float32),
        compiler_params=pltpu.CompilerParams(vmem_limit_bytes=100 * 1024 * 1024),
    )


@functools.lru_cache(maxsize=None)
def _gather_call(n_rows: int, table_rows: int, chunk: int):
    assert n_rows % (_NW * chunk) == 0 and chunk % 8 == 0
    n_chunks = n_rows // (_NW * chunk)
    rows_per_w = n_chunks * chunk
    mesh = plsc.VectorSubcoreMesh(
        core_axis_name="c", subcore_axis_name="s",
        num_cores=_NC, num_subcores=_NS)

    @functools.partial(
        pl.kernel,
        out_type=jax.ShapeDtypeStruct((n_rows, _D), jnp.float32),
        mesh=mesh,
        scratch_types=[
            pltpu.VMEM((chunk,), jnp.int32),
            pltpu.VMEM((chunk, _D), jnp.float32),
            pltpu.SemaphoreType.DMA,
        ],
        compiler_params=pltpu.CompilerParams(use_tc_tiling_on_sc=False),
    )
    def k(idx_hbm, table_hbm, out_hbm, idx_v, rows_v, sem):
        wid = lax.axis_index("s") * _NC + lax.axis_index("c")
        base0 = wid * rows_per_w

        def body(i, carry):
            base = base0 + i * chunk
            pltpu.sync_copy(idx_hbm.at[pl.ds(base, chunk)], idx_v)
            pltpu.async_copy(table_hbm.at[idx_v], rows_v, sem).wait()
            pltpu.sync_copy(rows_v, out_hbm.at[pl.ds(base, chunk)])
            return carry

        lax.fori_loop(0, n_chunks, body, 0, unroll=False)

    return k


def _lnt_body(x_ref, pe_ref, g_ref, b_ref, o_ref):
    s2_len = pe_ref.shape[1]
    x = x_ref[...].reshape(128, s2_len, 2 * _D) + pe_ref[...]
    lane = lax.broadcasted_iota(jnp.int32, (2 * _D, 2), 0)
    col = lax.broadcasted_iota(jnp.int32, (2 * _D, 2), 1)
    sel = (lane // _D) == col
    m = jnp.where(sel, 1.0 / _D, 0.0)
    p = jnp.where(sel, 1.0, 0.0).T
    mu = jnp.dot(jnp.dot(x, m, preferred_element_type=jnp.float32), p,
                 preferred_element_type=jnp.float32)
    xc = x - mu
    var = jnp.dot(jnp.dot(xc * xc, m, preferred_element_type=jnp.float32), p,
                  preferred_element_type=jnp.float32)
    y = xc * lax.rsqrt(var + _EPS) * g_ref[...] + b_ref[...]
    for s2 in range(s2_len):
        t = y[:, s2, :].T.reshape(2, 8, 8, 128)
        o_ref[2 * s2, :, 0, :, :] = t[0]
        o_ref[2 * s2 + 1, :, 0, :, :] = t[1]


@functools.lru_cache(maxsize=None)
def _lnt_call(batch: int, seq: int):
    s2 = seq // 2
    grid = (batch // 128,)
    return pl.pallas_call(
        _lnt_body,
        grid=grid,
        in_specs=[
            pl.BlockSpec((128 * s2, 2 * _D), lambda i: (i, 0)),
            pl.BlockSpec((1, s2, 2 * _D), lambda i: (0, 0, 0)),
            pl.BlockSpec((1, 1, 2 * _D), lambda i: (0, 0, 0)),
            pl.BlockSpec((1, 1, 2 * _D), lambda i: (0, 0, 0)),
        ],
        out_specs=pl.BlockSpec((seq, 8, 1, 8, 128), lambda i: (0, 0, i, 0, 0)),
        out_shape=jax.ShapeDtypeStruct(
            (seq, 8, batch // 128, 8, 128), jnp.float32),
        compiler_params=pltpu.CompilerParams(vmem_limit_bytes=100 * 1024 * 1024),
    )


def kernel(input_ids, table, gamma, beta):
    B, S = input_ids.shape
    n = B * S
    V = table.shape[0]
    half = ((V // 2) // _TW + 1) * _TW

    t2 = _tr_call(V, half)(table.T, table.T)
    t_lin = t2.reshape(2 * half, _D)

    ids = input_ids.reshape(-1).astype(jnp.int32)
    idx = jnp.where(ids < half, 2 * ids, 2 * (ids - half) + 1)

    emb = _gather_call(n, 2 * half, 800)(idx, t_lin)
    emb2 = emb.reshape(n // 2, 2 * _D)

    pe3 = jnp.asarray(_pe_const(S)).reshape(1, S // 2, 2 * _D)
    g2 = jnp.concatenate([gamma, gamma]).reshape(1, 1, 2 * _D)
    b2 = jnp.concatenate([beta, beta]).reshape(1, 1, 2 * _D)
    out_t = _lnt_call(B, S)(emb2, pe3, g2, b2)
    return out_t.transpose(2, 4, 0, 1, 3).reshape(B, S, _D)


# speedup vs baseline: 1.7875x; 1.0162x over previous
import functools

import numpy as np
import jax
import jax.numpy as jnp
from jax import lax
from jax.experimental import pallas as pl
from jax.experimental.pallas import tpu as pltpu
from jax.experimental.pallas import tpu_sc as plsc

_D = 64
_NC, _NS = 2, 16
_NW = _NC * _NS
_EPS = 1e-12
_TW = 16384


@functools.lru_cache(maxsize=None)
def _pe_const(seq_len: int):
    position = np.arange(0, seq_len, dtype=np.float32)[:, None]
    div_term = np.exp(np.arange(0, _D, 2, dtype=np.float32) * -(np.log(10000.0) / _D))
    pe = np.zeros((seq_len, _D), dtype=np.float32)
    pe[:, 0::2] = np.sin(position * div_term)
    pe[:, 1::2] = np.cos(position * div_term)
    return pe


def _tr_body(x1_ref, x2_ref, o_ref):
    o_ref[...] = jnp.concatenate([x1_ref[...].T, x2_ref[...].T], axis=1)


@functools.lru_cache(maxsize=None)
def _tr_call(vocab: int, half: int):
    grid = half // _TW
    last = (vocab - 1) // _TW
    return pl.pallas_call(
        _tr_body,
        grid=(grid,),
        in_specs=[
            pl.BlockSpec((_D, _TW), lambda i: (0, i)),
            pl.BlockSpec((_D, _TW), lambda i: (0, jnp.minimum(i + grid, last))),
        ],
        out_specs=pl.BlockSpec((_TW, 2 * _D), lambda i: (i, 0)),
        out_shape=jax.ShapeDtypeStruct((half, 2 * _D), jnp.float32),
        compiler_params=pltpu.CompilerParams(vmem_limit_bytes=100 * 1024 * 1024),
    )


@functools.lru_cache(maxsize=None)
def _gather_call(n_rows: int, table_rows: int, chunk: int):
    assert n_rows % (_NW * chunk) == 0 and chunk % 8 == 0
    n_chunks = n_rows // (_NW * chunk)
    rows_per_w = n_chunks * chunk
    mesh = plsc.VectorSubcoreMesh(
        core_axis_name="c", subcore_axis_name="s",
        num_cores=_NC, num_subcores=_NS)

    @functools.partial(
        pl.kernel,
        out_type=jax.ShapeDtypeStruct((n_rows, _D), jnp.float32),
        mesh=mesh,
        scratch_types=[
            pltpu.VMEM((chunk,), jnp.int32),
            pltpu.VMEM((chunk, _D), jnp.float32),
            pltpu.SemaphoreType.DMA,
        ],
        compiler_params=pltpu.CompilerParams(use_tc_tiling_on_sc=False),
    )
    def k(idx_hbm, table_hbm, out_hbm, idx_v, rows_v, sem):
        wid = lax.axis_index("s") * _NC + lax.axis_index("c")
        base0 = wid * rows_per_w

        def body(i, carry):
            base = base0 + i * chunk
            pltpu.sync_copy(idx_hbm.at[pl.ds(base, chunk)], idx_v)
            pltpu.async_copy(table_hbm.at[idx_v], rows_v, sem).wait()
            pltpu.sync_copy(rows_v, out_hbm.at[pl.ds(base, chunk)])
            return carry

        lax.fori_loop(0, n_chunks, body, 0, unroll=False)

    return k


def _lnt_body(x_ref, pe_ref, g_ref, b_ref, o_ref):
    s2_len = pe_ref.shape[1]
    x = x_ref[...].reshape(128, s2_len, 2 * _D) + pe_ref[...]
    lane = lax.broadcasted_iota(jnp.int32, (2 * _D, 2), 0)
    col = lax.broadcasted_iota(jnp.int32, (2 * _D, 2), 1)
    sel = (lane // _D) == col
    m = jnp.where(sel, 1.0 / _D, 0.0)
    p = jnp.where(sel, 1.0, 0.0).T
    mu = jnp.dot(jnp.dot(x, m, preferred_element_type=jnp.float32), p,
                 preferred_element_type=jnp.float32)
    xc = x - mu
    var = jnp.dot(jnp.dot(xc * xc, m, preferred_element_type=jnp.float32), p,
                  preferred_element_type=jnp.float32)
    y = xc * lax.rsqrt(var + _EPS) * g_ref[...] + b_ref[...]
    for s2 in range(s2_len):
        t = y[:, s2, :].T.reshape(2, 8, 8, 128)
        o_ref[2 * s2, :, 0, :, :] = t[0]
        o_ref[2 * s2 + 1, :, 0, :, :] = t[1]


@functools.lru_cache(maxsize=None)
def _lnt_call(batch: int, seq: int):
    s2 = seq // 2
    grid = (batch // 128,)
    return pl.pallas_call(
        _lnt_body,
        grid=grid,
        in_specs=[
            pl.BlockSpec((128 * s2, 2 * _D), lambda i: (i, 0)),
            pl.BlockSpec((1, s2, 2 * _D), lambda i: (0, 0, 0)),
            pl.BlockSpec((1, 1, 2 * _D), lambda i: (0, 0, 0)),
            pl.BlockSpec((1, 1, 2 * _D), lambda i: (0, 0, 0)),
        ],
        out_specs=pl.BlockSpec((seq, 8, 1, 8, 128), lambda i: (0, 0, i, 0, 0)),
        out_shape=jax.ShapeDtypeStruct(
            (seq, 8, batch // 128, 8, 128), jnp.float32),
        compiler_params=pltpu.CompilerParams(vmem_limit_bytes=100 * 1024 * 1024),
    )


def kernel(input_ids, table, gamma, beta):
    B, S = input_ids.shape
    n = B * S
    V = table.shape[0]
    half = ((V // 2) // _TW + 1) * _TW

    t2 = _tr_call(V, half)(table.T, table.T)
    t_lin = t2.reshape(2 * half, _D)

    ids = input_ids.reshape(-1).astype(jnp.int32)
    idx = jnp.where(ids < half, 2 * ids, 2 * (ids - half) + 1)

    emb = _gather_call(n, 2 * half, 1600)(idx, t_lin)
    emb2 = emb.reshape(n // 2, 2 * _D)

    pe3 = jnp.asarray(_pe_const(S)).reshape(1, S // 2, 2 * _D)
    g2 = jnp.concatenate([gamma, gamma]).reshape(1, 1, 2 * _D)
    b2 = jnp.concatenate([beta, beta]).reshape(1, 1, 2 * _D)
    out_t = _lnt_call(B, S)(emb2, pe3, g2, b2)
    return out_t.transpose(2, 4, 0, 1, 3).reshape(B, S, _D)
